# Initial kernel scaffold; baseline (speedup 1.0000x reference)
import jax
import jax.numpy as jnp
from jax.experimental import pallas as pl


def kernel(sequence, position_ids, token_table, pos_table):
    raise NotImplementedError("write your pallas kernel here")


# speedup vs baseline: 2.4677x; 2.4677x over previous
import functools

import jax
import jax.numpy as jnp
from jax import lax
from jax.experimental import pallas as pl
from jax.experimental.pallas import tpu as pltpu
from jax.experimental.pallas import tpu_sc as plsc

VOCAB = 1000000
D = 64
PMAX = 200
N = 4096 * 200
NC, NS = 2, 16
NW = NC * NS
PER_W = N // NW
CHUNK = 512
KSUB = CHUNK // 128
NCHUNK = PER_W // CHUNK
ROWS_W = PER_W // 128
LANES = 16
UNROLL = 4


def _body(seq_hbm, pid_hbm, tok_hbm, pos_hbm, out_hbm,
          idx_v, pidx_v, buf, pbuf, pos_sp, sem_t, sem_p):
    c = lax.axis_index("c")
    s = lax.axis_index("s")
    wid = s * NC + c

    @pl.when(s == 0)
    def _stage():
        pltpu.sync_copy(pos_hbm, pos_sp)

    plsc.subcore_barrier()

    def chunk_body(ci, carry):
        row0 = wid * ROWS_W + ci * KSUB
        pltpu.sync_copy(seq_hbm.at[pl.ds(row0, KSUB)], idx_v)
        pltpu.sync_copy(pid_hbm.at[pl.ds(row0, KSUB)], pidx_v)
        for k in range(KSUB):
            pltpu.async_copy(tok_hbm.at[idx_v.at[k]],
                             buf.at[pl.ds(k * 128, 128)], sem_t)
            pltpu.async_copy(pos_sp.at[pidx_v.at[k]],
                             pbuf.at[pl.ds(k * 128, 128)], sem_p)
        for k in range(KSUB):
            pltpu.make_async_copy(tok_hbm.at[idx_v.at[k]],
                                  buf.at[pl.ds(k * 128, 128)], sem_t).wait()
            pltpu.make_async_copy(pos_sp.at[pidx_v.at[k]],
                                  pbuf.at[pl.ds(k * 128, 128)], sem_p).wait()

        def add_body(i, acc):
            base = i * UNROLL
            for u in range(UNROLL):
                for j in range(D // LANES):
                    v = pbuf[base + u, pl.ds(j * LANES, LANES)]
                    plsc.addupdate(buf.at[base + u, pl.ds(j * LANES, LANES)], v)
            return acc

        lax.fori_loop(0, CHUNK // UNROLL, add_body, 0, unroll=False)
        pltpu.sync_copy(buf, out_hbm.at[pl.ds(wid * PER_W + ci * CHUNK, CHUNK)])
        return carry

    lax.fori_loop(0, NCHUNK, chunk_body, 0, unroll=False)


@jax.jit
def _embed_sum(seq2d, pid2d, token_table, pos_table):
    mesh = plsc.VectorSubcoreMesh(core_axis_name="c", subcore_axis_name="s")
    kern = pl.kernel(
        _body,
        out_type=jax.ShapeDtypeStruct((N, D), jnp.float32),
        mesh=mesh,
        scratch_types=[
            pltpu.VMEM((KSUB, 128), jnp.int32),
            pltpu.VMEM((KSUB, 128), jnp.int32),
            pltpu.VMEM((CHUNK, D), jnp.float32),
            pltpu.VMEM((CHUNK, D), jnp.float32),
            pltpu.VMEM_SHARED((PMAX, D), jnp.float32),
            pltpu.SemaphoreType.DMA,
            pltpu.SemaphoreType.DMA,
        ],
        compiler_params=pltpu.CompilerParams(use_tc_tiling_on_sc=False),
    )
    return kern(seq2d, pid2d, token_table, pos_table)


def kernel(sequence, position_ids, token_table, pos_table):
    b, s = sequence.shape
    seq2d = sequence.reshape(N // 128, 128).astype(jnp.int32)
    pid2d = position_ids.reshape(N // 128, 128).astype(jnp.int32)
    out = _embed_sum(seq2d, pid2d, token_table, pos_table)
    return out.reshape(b, s, D)


# speedup vs baseline: 2.5738x; 1.0430x over previous
import functools

import jax
import jax.numpy as jnp
from jax import lax
from jax.experimental import pallas as pl
from jax.experimental.pallas import tpu as pltpu
from jax.experimental.pallas import tpu_sc as plsc

VOCAB = 1000000
D = 64
PMAX = 200
N = 4096 * 200
NC, NS = 2, 16
NW = NC * NS
PER_W = N // NW
CHUNK = 512
KSUB = CHUNK // 128
NCHUNK = PER_W // CHUNK
ROWS_W = PER_W // 128
LANES = 16
UNROLL = 4


def _body(seq_hbm, pid_hbm, tok_hbm, pos_hbm, out_hbm,
          idx_v, pidx_v, buf, pbuf, pos_sp, sem_t, sem_p):
    c = lax.axis_index("c")
    s = lax.axis_index("s")
    wid = s * NC + c

    @pl.when(s == 0)
    def _stage():
        pltpu.sync_copy(pos_hbm, pos_sp)

    plsc.subcore_barrier()

    def chunk_body(ci, carry):
        row0 = wid * ROWS_W + ci * KSUB
        pltpu.sync_copy(seq_hbm.at[pl.ds(row0, KSUB)], idx_v)
        pltpu.sync_copy(pid_hbm.at[pl.ds(row0, KSUB)], pidx_v)
        for k in range(KSUB):
            pltpu.async_copy(pos_sp.at[pidx_v.at[k]],
                             buf.at[pl.ds(k * 128, 128)], sem_p)
        for k in range(KSUB):
            pltpu.make_async_copy(pos_sp.at[pidx_v.at[k]],
                                  buf.at[pl.ds(k * 128, 128)], sem_p).wait()
        for k in range(KSUB):
            pltpu.async_copy(tok_hbm.at[idx_v.at[k]],
                             buf.at[pl.ds(k * 128, 128)], sem_t, add=True)
        for k in range(KSUB):
            pltpu.make_async_copy(tok_hbm.at[idx_v.at[k]],
                                  buf.at[pl.ds(k * 128, 128)], sem_t).wait()
        pltpu.sync_copy(buf, out_hbm.at[pl.ds(wid * PER_W + ci * CHUNK, CHUNK)])
        return carry

    lax.fori_loop(0, NCHUNK, chunk_body, 0, unroll=False)


@jax.jit
def _embed_sum(seq2d, pid2d, token_table, pos_table):
    mesh = plsc.VectorSubcoreMesh(core_axis_name="c", subcore_axis_name="s")
    kern = pl.kernel(
        _body,
        out_type=jax.ShapeDtypeStruct((N, D), jnp.float32),
        mesh=mesh,
        scratch_types=[
            pltpu.VMEM((KSUB, 128), jnp.int32),
            pltpu.VMEM((KSUB, 128), jnp.int32),
            pltpu.VMEM((CHUNK, D), jnp.float32),
            pltpu.VMEM((CHUNK, D), jnp.float32),
            pltpu.VMEM_SHARED((PMAX, D), jnp.float32),
            pltpu.SemaphoreType.DMA,
            pltpu.SemaphoreType.DMA,
        ],
        compiler_params=pltpu.CompilerParams(use_tc_tiling_on_sc=False),
    )
    return kern(seq2d, pid2d, token_table, pos_table)


def kernel(sequence, position_ids, token_table, pos_table):
    b, s = sequence.shape
    seq2d = sequence.reshape(N // 128, 128).astype(jnp.int32)
    pid2d = position_ids.reshape(N // 128, 128).astype(jnp.int32)
    out = _embed_sum(seq2d, pid2d, token_table, pos_table)
    return out.reshape(b, s, D)
